# baseline (speedup 1.0000x reference)
import functools

import jax
import jax.numpy as jnp
from jax import lax
from jax.experimental import pallas as pl
from jax.experimental.pallas import tpu as pltpu

PAD_ROW = 0
V = 1000
VP = 1024
H = 64
B = 1024
L = 50
LP = 56
BT = 32
GRID = B // BT


def _proj_body(emb_ref, w_ref, b_ref, out_ref):
    emb = emb_ref[:]
    rows = lax.broadcasted_iota(jnp.int32, emb.shape, 0)
    emb = jnp.where(rows == PAD_ROW, 0.0, emb)
    acc = lax.dot_general(
        emb, w_ref[:], (((1,), (1,)), ((), ())),
        preferred_element_type=jnp.float32,
    )
    out_ref[:] = jnp.zeros((VP, V), jnp.bfloat16)
    out_ref[pl.ds(0, V), :] = (acc + b_ref[:]).astype(jnp.bfloat16)


def _logit_table(emb, w, b):
    return pl.pallas_call(
        _proj_body,
        out_shape=jax.ShapeDtypeStruct((VP, V), jnp.bfloat16),
    )(emb, w, b.reshape(1, V))


def _onehot_body(ids_ref, lt_ref, out_ref):
    ids = ids_ref[:]
    vocab = lax.broadcasted_iota(jnp.int32, (BT, L, VP), 2)
    onehot = (ids[:, :, None] == vocab).astype(jnp.bfloat16)
    out_ref[:] = lax.dot_general(
        onehot, lt_ref[:], (((2,), (0,)), ((), ())),
        preferred_element_type=jnp.float32,
    )


def _onehot_gather(ids, ltb):
    return pl.pallas_call(
        _onehot_body,
        grid=(GRID,),
        in_specs=[
            pl.BlockSpec((BT, L), lambda i: (i, 0)),
            pl.BlockSpec((VP, V), lambda i: (0, 0)),
        ],
        out_specs=pl.BlockSpec((BT, L, V), lambda i: (i, 0, 0)),
        out_shape=jax.ShapeDtypeStruct((B, L, V), jnp.float32),
    )(ids, ltb)


def kernel(token_ids, emb_table, W, b):
    return _onehot_gather(token_ids, _logit_table(emb_table, W, b))


# speedup vs baseline: 1.0549x; 1.0549x over previous
import functools

import jax
import jax.numpy as jnp
from jax import lax
from jax.experimental import pallas as pl
from jax.experimental.pallas import tpu as pltpu

PAD_ROW = 0
V = 1000
VP = 1024
H = 64
B = 1024
L = 50
LP = 56
BT = 32
GRID = B // BT


def _proj_body(emb_ref, w_ref, b_ref, out_ref):
    emb = emb_ref[:]
    rows = lax.broadcasted_iota(jnp.int32, emb.shape, 0)
    emb = jnp.where(rows == PAD_ROW, 0.0, emb)
    acc = lax.dot_general(
        emb, w_ref[:], (((1,), (1,)), ((), ())),
        preferred_element_type=jnp.float32,
    )
    out_ref[:] = acc + b_ref[:]


def _logit_table(emb, w, b):
    return pl.pallas_call(
        _proj_body,
        out_shape=jax.ShapeDtypeStruct((V, V), jnp.float32),
    )(emb, w, b.reshape(1, V))


def _onehot_body(ids_ref, lt_ref, out_ref):
    ids = ids_ref[:]
    vocab = lax.broadcasted_iota(jnp.int32, (BT, LP, VP), 2)
    onehot = (ids[:, :, None] == vocab).astype(jnp.bfloat16)
    acc = lax.dot_general(
        onehot, lt_ref[:], (((2,), (0,)), ((), ())),
        preferred_element_type=jnp.float32,
    )
    out_ref[:] = acc[:, :L, :]


def _onehot_gather(ids_pad, ltb):
    return pl.pallas_call(
        _onehot_body,
        grid=(GRID,),
        in_specs=[
            pl.BlockSpec((BT, LP), lambda i: (i, 0)),
            pl.BlockSpec((VP, V), lambda i: (0, 0)),
        ],
        out_specs=pl.BlockSpec((BT, L, V), lambda i: (i, 0, 0)),
        out_shape=jax.ShapeDtypeStruct((B, L, V), jnp.float32),
    )(ids_pad, ltb)


def kernel(token_ids, emb_table, W, b):
    lt = _logit_table(emb_table, W, b)
    ltb = jnp.concatenate(
        [lt, jnp.zeros((VP - V, V), lt.dtype)], axis=0).astype(jnp.bfloat16)
    ids_pad = jnp.pad(token_ids, ((0, 0), (0, LP - L)))
    return _onehot_gather(ids_pad, ltb)


# speedup vs baseline: 1.0572x; 1.0021x over previous
import functools

import jax
import jax.numpy as jnp
from jax import lax
from jax.experimental import pallas as pl
from jax.experimental.pallas import tpu as pltpu

PAD_ROW = 0
V = 1000
VP = 1024
H = 64
B = 1024
L = 50
LP = 56
BT = 64
GRID = B // BT


def _proj_body(emb_ref, w_ref, b_ref, out_ref):
    emb = emb_ref[:]
    rows = lax.broadcasted_iota(jnp.int32, emb.shape, 0)
    emb = jnp.where(rows == PAD_ROW, 0.0, emb)
    acc = lax.dot_general(
        emb, w_ref[:], (((1,), (1,)), ((), ())),
        preferred_element_type=jnp.float32,
    )
    out_ref[:] = acc + b_ref[:]


def _logit_table(emb, w, b):
    return pl.pallas_call(
        _proj_body,
        out_shape=jax.ShapeDtypeStruct((V, V), jnp.float32),
    )(emb, w, b.reshape(1, V))


def _onehot_body(ids_ref, lt_ref, out_ref):
    ids = ids_ref[:]
    vocab = lax.broadcasted_iota(jnp.int32, (BT, LP, VP), 2)
    onehot = (ids[:, :, None] == vocab).astype(jnp.bfloat16)
    acc = lax.dot_general(
        onehot, lt_ref[:], (((2,), (0,)), ((), ())),
        preferred_element_type=jnp.float32,
    )
    out_ref[:] = acc[:, :L, :]


def _onehot_gather(ids_pad, ltb):
    return pl.pallas_call(
        _onehot_body,
        grid=(GRID,),
        in_specs=[
            pl.BlockSpec((BT, LP), lambda i: (i, 0)),
            pl.BlockSpec((VP, V), lambda i: (0, 0)),
        ],
        out_specs=pl.BlockSpec((BT, L, V), lambda i: (i, 0, 0)),
        out_shape=jax.ShapeDtypeStruct((B, L, V), jnp.float32),
    )(ids_pad, ltb)


def kernel(token_ids, emb_table, W, b):
    lt = _logit_table(emb_table, W, b)
    ltb = jnp.concatenate(
        [lt, jnp.zeros((VP - V, V), lt.dtype)], axis=0).astype(jnp.bfloat16)
    ids_pad = jnp.pad(token_ids, ((0, 0), (0, LP - L)))
    return _onehot_gather(ids_pad, ltb)


# speedup vs baseline: 1.0722x; 1.0143x over previous
import functools

import jax
import jax.numpy as jnp
from jax import lax
from jax.experimental import pallas as pl
from jax.experimental.pallas import tpu as pltpu

PAD_ROW = 0
V = 1000
VP = 1024
H = 64
B = 1024
L = 50
LP = 56
BT = 64
GRID = B // BT


def _proj_body(emb_ref, w_ref, b_ref, out_ref):
    emb = emb_ref[:]
    rows = lax.broadcasted_iota(jnp.int32, emb.shape, 0)
    emb = jnp.where(rows == PAD_ROW, 0.0, emb)
    acc = lax.dot_general(
        emb, w_ref[:], (((1,), (1,)), ((), ())),
        preferred_element_type=jnp.float32,
    )
    out_ref[:] = jnp.zeros((VP, V), jnp.bfloat16)
    out_ref[pl.ds(0, V), :] = (acc + b_ref[:]).astype(jnp.bfloat16)


def _logit_table(emb, w, b):
    return pl.pallas_call(
        _proj_body,
        out_shape=jax.ShapeDtypeStruct((VP, V), jnp.bfloat16),
    )(emb, w, b.reshape(1, V))


def _onehot_body(ids_ref, lt_ref, out_ref):
    ids = ids_ref[:]
    vocab = lax.broadcasted_iota(jnp.int32, (BT, LP, VP), 2)
    onehot = (ids[:, :, None] == vocab).astype(jnp.bfloat16)
    acc = lax.dot_general(
        onehot, lt_ref[:], (((2,), (0,)), ((), ())),
        preferred_element_type=jnp.float32,
    )
    out_ref[:] = acc[:, :L, :]


def _onehot_gather(ids_pad, ltb):
    return pl.pallas_call(
        _onehot_body,
        grid=(GRID,),
        in_specs=[
            pl.BlockSpec((BT, LP), lambda i: (i, 0)),
            pl.BlockSpec((VP, V), lambda i: (0, 0)),
        ],
        out_specs=pl.BlockSpec((BT, L, V), lambda i: (i, 0, 0)),
        out_shape=jax.ShapeDtypeStruct((B, L, V), jnp.float32),
    )(ids_pad, ltb)


def kernel(token_ids, emb_table, W, b):
    ltb = _logit_table(emb_table, W, b)
    ids_pad = jnp.pad(token_ids, ((0, 0), (0, LP - L)))
    return _onehot_gather(ids_pad, ltb)
